# baseline (speedup 1.0000x reference)
import jax
import jax.numpy as jnp
from jax import lax
from jax.experimental import pallas as pl
from jax.experimental.pallas import tpu as pltpu
from jax.experimental.pallas import tpu_sc as plsc

_C0 = 10000
_C1 = 100
_D = 32
_BATCH = 16384
_PK = 128 // _D

_NC = 2
_NS = 16
_NW = _NC * _NS
_BPW = _BATCH // _NW
_CHUNK = 128
_NCHUNK = _BPW // _CHUNK
_L = 16
_GPC = _CHUNK // _L


def _sc_body(uids_hbm, iids_hbm,
             user_obj, user_c0, user_c1,
             item_obj, item_c0, item_c1,
             out_hbm,
             uid_v, iid_v, uq_v, iq_v, uc0q_v, ic0q_v,
             uo_b, uc0_b, io_b, ic0_b,
             uc1_v, ic1_v,
             out_v, sem):
    wid = lax.axis_index("s") * _NC + lax.axis_index("c")
    base = wid * _BPW

    pltpu.sync_copy(uids_hbm.at[pl.ds(base, _BPW)], uid_v)
    pltpu.sync_copy(iids_hbm.at[pl.ds(base, _BPW)], iid_v)
    pltpu.sync_copy(user_c1, uc1_v)
    pltpu.sync_copy(item_c1, ic1_v)

    def _idx_body(g, _):
        sl = pl.ds(g * _L, _L)
        u = uid_v[sl]
        i = iid_v[sl]
        uq_v[sl] = lax.shift_right_logical(u, 2)
        iq_v[sl] = lax.shift_right_logical(i, 2)
        uc0q_v[sl] = lax.shift_right_logical(lax.rem(u, _C0), 2)
        ic0q_v[sl] = lax.shift_right_logical(lax.rem(i, _C0), 2)
        return 0

    lax.fori_loop(0, _BPW // _L, _idx_body, 0)

    lanes = lax.iota(jnp.int32, _L)

    for c in range(_NCHUNK):
        csl = pl.ds(c * _CHUNK, _CHUNK)
        copies = [
            pltpu.make_async_copy(user_obj.at[uq_v.at[csl]], uo_b, sem),
            pltpu.make_async_copy(user_c0.at[uc0q_v.at[csl]], uc0_b, sem),
            pltpu.make_async_copy(item_obj.at[iq_v.at[csl]], io_b, sem),
            pltpu.make_async_copy(item_c0.at[ic0q_v.at[csl]], ic0_b, sem),
        ]
        for cp in copies:
            cp.start()
        for cp in copies:
            cp.wait()

        def _dot_body(g, _):
            gsl = pl.ds(c * _CHUNK + g * _L, _L)
            uvec = uid_v[gsl]
            ivec = iid_v[gsl]
            uoff = (uvec & (_PK - 1)) * _D
            ioff = (ivec & (_PK - 1)) * _D
            uq1 = lax.shift_right_logical(lax.rem(uvec, _C1), 2)
            iq1 = lax.shift_right_logical(lax.rem(ivec, _C1), 2)
            acc = jnp.zeros((_L,), jnp.float32)
            for r16 in range(_L):
                r = g * _L + r16
                uo = uoff[r16]
                io = ioff[r16]
                p = jnp.zeros((_L,), jnp.float32)
                for h in range(_D // _L):
                    us = pl.ds(uo + h * _L, _L)
                    vs = pl.ds(io + h * _L, _L)
                    u = (uo_b[r, us] + uc0_b[r, us]
                         + uc1_v[uq1[r16], us])
                    v = (io_b[r, vs] + ic0_b[r, vs]
                         + ic1_v[iq1[r16], vs])
                    p = p + u * v
                acc = jnp.where(lanes == r16, jnp.sum(p), acc)
            out_v[gsl] = acc
            return 0

        lax.fori_loop(0, _GPC, _dot_body, 0)

    pltpu.sync_copy(out_v, out_hbm.at[pl.ds(base, _BPW)])


def kernel(X, user_obj, user_c0, user_c1, item_obj, item_c0, item_c1):
    uids = X[:, 0]
    iids = X[:, 1]

    uo128 = user_obj.reshape(-1, 128)
    uc0128 = user_c0.reshape(-1, 128)
    uc1128 = user_c1.reshape(-1, 128)
    io128 = item_obj.reshape(-1, 128)
    ic0128 = item_c0.reshape(-1, 128)
    ic1128 = item_c1.reshape(-1, 128)

    mesh = plsc.VectorSubcoreMesh(core_axis_name="c", subcore_axis_name="s")
    k = pl.kernel(
        _sc_body,
        out_type=jax.ShapeDtypeStruct((_BATCH,), jnp.float32),
        mesh=mesh,
        compiler_params=pltpu.CompilerParams(needs_layout_passes=False),
        scratch_types=[
            pltpu.VMEM((_BPW,), jnp.int32),
            pltpu.VMEM((_BPW,), jnp.int32),
            pltpu.VMEM((_BPW,), jnp.int32),
            pltpu.VMEM((_BPW,), jnp.int32),
            pltpu.VMEM((_BPW,), jnp.int32),
            pltpu.VMEM((_BPW,), jnp.int32),
            pltpu.VMEM((_CHUNK, 128), jnp.float32),
            pltpu.VMEM((_CHUNK, 128), jnp.float32),
            pltpu.VMEM((_CHUNK, 128), jnp.float32),
            pltpu.VMEM((_CHUNK, 128), jnp.float32),
            pltpu.VMEM((_C1 // _PK, 128), jnp.float32),
            pltpu.VMEM((_C1 // _PK, 128), jnp.float32),
            pltpu.VMEM((_BPW,), jnp.float32),
            pltpu.SemaphoreType.DMA,
        ],
    )
    out = k(uids, iids, uo128, uc0128, uc1128, io128, ic0128, ic1128)
    return out.reshape(_BATCH, 1)
